# Initial kernel scaffold; baseline (speedup 1.0000x reference)
import functools

import numpy as np
import jax
import jax.numpy as jnp
from jax.experimental import pallas as pl
from jax.experimental.pallas import tpu as pltpu

_K = 8192
_FEAT = 256
_NCLS = 65
_B = 256
_DIN = 2048
_M = 0.999
_T = 0.07

_TJ = 512


@functools.lru_cache(maxsize=1)
def _perm_np():
    return np.asarray(jax.random.permutation(jax.random.key(1), _B),
                      dtype=np.int32)


def _k1_body(im_q_ref, im_k_ref, W_q_ref, b_q_ref, W_cls_ref, b_cls_ref,
             W_k_ref, b_k_ref, W_cls_k_ref, b_cls_k_ref,
             feats_q_ref, logits_q_ref, q_ref, k_ref, kT_ref,
             logits_k_ref, l_pos_ref, probs_ref):
    im_q = im_q_ref[...]
    W_q = W_q_ref[...]
    b_q = b_q_ref[...]
    feats_q = jnp.dot(im_q, W_q, preferred_element_type=jnp.float32) + b_q
    feats_q_ref[...] = feats_q
    logits_q_ref[...] = (
        jnp.dot(feats_q, W_cls_ref[...], preferred_element_type=jnp.float32)
        + b_cls_ref[...])
    nq = jnp.sqrt(jnp.sum(feats_q * feats_q, axis=1, keepdims=True))
    q = feats_q / jnp.maximum(nq, 1e-12)
    q_ref[...] = q

    W_k2 = W_k_ref[...] * _M + W_q * (1.0 - _M)
    b_k2 = b_k_ref[...] * _M + b_q * (1.0 - _M)
    W_cls_k2 = W_cls_k_ref[...] * _M + W_cls_ref[...] * (1.0 - _M)
    b_cls_k2 = b_cls_k_ref[...] * _M + b_cls_ref[...] * (1.0 - _M)

    feats_k = jnp.dot(im_k_ref[...], W_k2, preferred_element_type=jnp.float32) + b_k2
    logits_k = (jnp.dot(feats_k, W_cls_k2, preferred_element_type=jnp.float32)
                + b_cls_k2)
    logits_k_ref[...] = logits_k
    nk = jnp.sqrt(jnp.sum(feats_k * feats_k, axis=1, keepdims=True))
    k = feats_k / jnp.maximum(nk, 1e-12)
    k_ref[...] = k
    kT_ref[...] = k.T

    l_pos_ref[...] = jnp.sum(q * k, axis=1, keepdims=True) * (1.0 / _T)

    m = jnp.max(logits_k, axis=1, keepdims=True)
    e = jnp.exp(logits_k - m)
    probs_ref[...] = e / jnp.sum(e, axis=1, keepdims=True)


def _k2_body(memf_ref, q_ref, lnn_ref, lneg_ref):
    j = pl.program_id(0)
    a = memf_ref[...]
    b = memf_ref[:, pl.ds(j * _TJ, _TJ)]
    lnn_ref[...] = jax.lax.dot_general(
        a, b, (((0,), (0,)), ((), ())), preferred_element_type=jnp.float32)
    lneg_ref[...] = jnp.dot(q_ref[...], b,
                            preferred_element_type=jnp.float32) * (1.0 / _T)


def _k3_body(memf_ref, kT_ref, memp_ref, probs_ref, meml_ref, pseudo_ref,
             memi_ref, idxs_ref, memf_out, memp_out, meml_out, memi_out):
    memf_out[...] = memf_ref[...]
    memf_out[:, 0:_B] = kT_ref[...]
    memp_out[...] = memp_ref[...]
    memp_out[0:_B, :] = probs_ref[...]
    meml_out[...] = meml_ref[...]
    meml_out[0:2, :] = pseudo_ref[...]
    memi_out[...] = memi_ref[...]
    memi_out[0:2, :] = idxs_ref[...]


def kernel(im_q, im_k, idxs, W_q, b_q, W_cls, b_cls, W_k, b_k, W_cls_k,
           b_cls_k, mem_feat, mem_labels, mem_probs, mem_index):
    perm = _perm_np()
    f32 = jnp.float32

    (feats_q, logits_q, q, k, kT, logits_k_u, l_pos, probs_u) = pl.pallas_call(
        _k1_body,
        out_shape=(
            jax.ShapeDtypeStruct((_B, _FEAT), f32),
            jax.ShapeDtypeStruct((_B, _NCLS), f32),
            jax.ShapeDtypeStruct((_B, _FEAT), f32),
            jax.ShapeDtypeStruct((_B, _FEAT), f32),
            jax.ShapeDtypeStruct((_FEAT, _B), f32),
            jax.ShapeDtypeStruct((_B, _NCLS), f32),
            jax.ShapeDtypeStruct((_B, 1), f32),
            jax.ShapeDtypeStruct((_B, _NCLS), f32),
        ),
    )(im_q, im_k, W_q, b_q.reshape(1, _FEAT), W_cls, b_cls.reshape(1, _NCLS),
      W_k, b_k.reshape(1, _FEAT), W_cls_k, b_cls_k.reshape(1, _NCLS))

    l_neg_near, l_neg = pl.pallas_call(
        _k2_body,
        grid=(_K // _TJ,),
        in_specs=[
            pl.BlockSpec((_FEAT, _K), lambda j: (0, 0)),
            pl.BlockSpec((_B, _FEAT), lambda j: (0, 0)),
        ],
        out_specs=[
            pl.BlockSpec((_K, _TJ), lambda j: (0, j)),
            pl.BlockSpec((_B, _TJ), lambda j: (0, j)),
        ],
        out_shape=(
            jax.ShapeDtypeStruct((_K, _K), f32),
            jax.ShapeDtypeStruct((_B, _K), f32),
        ),
    )(mem_feat, q)

    logits_k = logits_k_u[perm]
    probs = probs_u[perm]
    pseudo = jnp.argmax(logits_k, axis=1).astype(mem_labels.dtype)

    meml2d = mem_labels.reshape(_K // 128, 128)
    memi2d = mem_index.reshape(_K // 128, 128)
    mem_feat_new, mem_probs_new, meml_new, memi_new = pl.pallas_call(
        _k3_body,
        out_shape=(
            jax.ShapeDtypeStruct((_FEAT, _K), f32),
            jax.ShapeDtypeStruct((_K, _NCLS), f32),
            jax.ShapeDtypeStruct((_K // 128, 128), mem_labels.dtype),
            jax.ShapeDtypeStruct((_K // 128, 128), mem_index.dtype),
        ),
    )(mem_feat, kT, mem_probs, probs, meml2d, pseudo.reshape(2, 128),
      memi2d, idxs.astype(mem_index.dtype).reshape(2, 128))

    logits_ins = jnp.concatenate([l_pos, l_neg], axis=1)
    return (feats_q, logits_q, logits_ins, k, logits_k, l_neg_near,
            mem_feat_new, meml_new.reshape(_K), mem_probs_new,
            memi_new.reshape(_K))


# speedup vs baseline: 1.1643x; 1.1643x over previous
import functools

import numpy as np
import jax
import jax.numpy as jnp
from jax.experimental import pallas as pl
from jax.experimental.pallas import tpu as pltpu

_K = 8192
_FEAT = 256
_NCLS = 65
_B = 256
_DIN = 2048
_M = 0.999
_T = 0.07

_TJ = 512


@functools.lru_cache(maxsize=1)
def _perm_np():
    with jax.ensure_compile_time_eval():
        p = jax.random.permutation(jax.random.key(1), _B)
        return np.asarray(p, dtype=np.int32)


def _k1_body(im_q_ref, im_k_ref, W_q_ref, b_q_ref, W_cls_ref, b_cls_ref,
             W_k_ref, b_k_ref, W_cls_k_ref, b_cls_k_ref,
             feats_q_ref, logits_q_ref, q_ref, k_ref, kT_ref,
             logits_k_ref, l_pos_ref, probs_ref):
    im_q = im_q_ref[...]
    W_q = W_q_ref[...]
    b_q = b_q_ref[...]
    feats_q = jnp.dot(im_q, W_q, preferred_element_type=jnp.float32) + b_q
    feats_q_ref[...] = feats_q
    logits_q_ref[...] = (
        jnp.dot(feats_q, W_cls_ref[...], preferred_element_type=jnp.float32)
        + b_cls_ref[...])
    nq = jnp.sqrt(jnp.sum(feats_q * feats_q, axis=1, keepdims=True))
    q = feats_q / jnp.maximum(nq, 1e-12)
    q_ref[...] = q

    W_k2 = W_k_ref[...] * _M + W_q * (1.0 - _M)
    b_k2 = b_k_ref[...] * _M + b_q * (1.0 - _M)
    W_cls_k2 = W_cls_k_ref[...] * _M + W_cls_ref[...] * (1.0 - _M)
    b_cls_k2 = b_cls_k_ref[...] * _M + b_cls_ref[...] * (1.0 - _M)

    feats_k = jnp.dot(im_k_ref[...], W_k2, preferred_element_type=jnp.float32) + b_k2
    logits_k = (jnp.dot(feats_k, W_cls_k2, preferred_element_type=jnp.float32)
                + b_cls_k2)
    logits_k_ref[...] = logits_k
    nk = jnp.sqrt(jnp.sum(feats_k * feats_k, axis=1, keepdims=True))
    k = feats_k / jnp.maximum(nk, 1e-12)
    k_ref[...] = k
    kT_ref[...] = k.T

    l_pos_ref[...] = jnp.sum(q * k, axis=1, keepdims=True) * (1.0 / _T)

    m = jnp.max(logits_k, axis=1, keepdims=True)
    e = jnp.exp(logits_k - m)
    probs_ref[...] = e / jnp.sum(e, axis=1, keepdims=True)


def _k2_body(memf_ref, q_ref, lnn_ref, lneg_ref):
    j = pl.program_id(0)
    a = memf_ref[...]
    b = memf_ref[:, pl.ds(j * _TJ, _TJ)]
    lnn_ref[...] = jax.lax.dot_general(
        a, b, (((0,), (0,)), ((), ())), preferred_element_type=jnp.float32)
    lneg_ref[...] = jnp.dot(q_ref[...], b,
                            preferred_element_type=jnp.float32) * (1.0 / _T)


def _k3_body(memf_ref, kT_ref, memp_ref, probs_ref, meml_ref, pseudo_ref,
             memi_ref, idxs_ref, memf_out, memp_out, meml_out, memi_out):
    memf_out[...] = memf_ref[...]
    memf_out[:, 0:_B] = kT_ref[...]
    memp_out[...] = memp_ref[...]
    memp_out[0:_B, :] = probs_ref[...]
    meml_out[...] = meml_ref[...]
    meml_out[0:2, :] = pseudo_ref[...]
    memi_out[...] = memi_ref[...]
    memi_out[0:2, :] = idxs_ref[...]


def kernel(im_q, im_k, idxs, W_q, b_q, W_cls, b_cls, W_k, b_k, W_cls_k,
           b_cls_k, mem_feat, mem_labels, mem_probs, mem_index):
    perm = _perm_np()
    f32 = jnp.float32

    (feats_q, logits_q, q, k, kT, logits_k_u, l_pos, probs_u) = pl.pallas_call(
        _k1_body,
        out_shape=(
            jax.ShapeDtypeStruct((_B, _FEAT), f32),
            jax.ShapeDtypeStruct((_B, _NCLS), f32),
            jax.ShapeDtypeStruct((_B, _FEAT), f32),
            jax.ShapeDtypeStruct((_B, _FEAT), f32),
            jax.ShapeDtypeStruct((_FEAT, _B), f32),
            jax.ShapeDtypeStruct((_B, _NCLS), f32),
            jax.ShapeDtypeStruct((_B, 1), f32),
            jax.ShapeDtypeStruct((_B, _NCLS), f32),
        ),
    )(im_q, im_k, W_q, b_q.reshape(1, _FEAT), W_cls, b_cls.reshape(1, _NCLS),
      W_k, b_k.reshape(1, _FEAT), W_cls_k, b_cls_k.reshape(1, _NCLS))

    l_neg_near, l_neg = pl.pallas_call(
        _k2_body,
        grid=(_K // _TJ,),
        in_specs=[
            pl.BlockSpec((_FEAT, _K), lambda j: (0, 0)),
            pl.BlockSpec((_B, _FEAT), lambda j: (0, 0)),
        ],
        out_specs=[
            pl.BlockSpec((_K, _TJ), lambda j: (0, j)),
            pl.BlockSpec((_B, _TJ), lambda j: (0, j)),
        ],
        out_shape=(
            jax.ShapeDtypeStruct((_K, _K), f32),
            jax.ShapeDtypeStruct((_B, _K), f32),
        ),
    )(mem_feat, q)

    logits_k = logits_k_u[perm]
    probs = probs_u[perm]
    pseudo = jnp.argmax(logits_k, axis=1).astype(mem_labels.dtype)

    meml2d = mem_labels.reshape(_K // 128, 128)
    memi2d = mem_index.reshape(_K // 128, 128)
    mem_feat_new, mem_probs_new, meml_new, memi_new = pl.pallas_call(
        _k3_body,
        out_shape=(
            jax.ShapeDtypeStruct((_FEAT, _K), f32),
            jax.ShapeDtypeStruct((_K, _NCLS), f32),
            jax.ShapeDtypeStruct((_K // 128, 128), mem_labels.dtype),
            jax.ShapeDtypeStruct((_K // 128, 128), mem_index.dtype),
        ),
    )(mem_feat, kT, mem_probs, probs, meml2d, pseudo.reshape(2, 128),
      memi2d, idxs.astype(mem_index.dtype).reshape(2, 128))

    logits_ins = jnp.concatenate([l_pos, l_neg], axis=1)
    return (feats_q, logits_q, logits_ins, k, logits_k, l_neg_near,
            mem_feat_new, meml_new.reshape(_K), mem_probs_new,
            memi_new.reshape(_K))
